# baseline (speedup 1.0000x reference)
import jax
import jax.numpy as jnp
from jax import lax
from jax.experimental import pallas as pl
from jax.experimental.pallas import tpu as pltpu
from jax.experimental.pallas import tpu_sc as plsc

BATCH = 16384
HIST = 50
DIM = 64
NB = 128
NBLK = HIST * (BATCH // NB)
NC = 2
NS = 16
NW = NC * NS
BLK_PER_W = NBLK // NW
L = 16
TSTRIDE = NB + 1
NSETS = 4
NROWS = 1000000
FULL_TILES = NROWS // NB
TAIL = NROWS - FULL_TILES * NB
BSTRIDE = DIM + 1


def _build_body(tabt_hbm, tail_hbm, out_hbm, slab_v, t2p_v, tl_v, rsem, wsem):
    wid = lax.axis_index("s") * NC + lax.axis_index("c")
    extra = jnp.minimum(wid, 4)
    n_w = 244 + jnp.where(wid < 4, 1, 0)
    t0 = wid * 244 + extra

    iota = lax.iota(jnp.int32, L)
    pidx = [(iota + k * L) // 2 for k in range(NB // L)]
    hofs = [((iota + k * L) % 2) * DIM for k in range(NB // L)]

    def fire(t, s):
        pltpu.async_copy(
            tabt_hbm.at[:, pl.ds(t * NB, NB)], slab_v.at[s], rsem.at[s]
        )

    def drain_write(s):
        pltpu.make_async_copy(
            t2p_v.at[s], out_hbm.at[pl.ds(0, NB // 2)], wsem.at[s]
        ).wait()

    def transpose(s):
        @pl.loop(0, DIM, unroll=4)
        def _j(j):
            jb = jnp.full((L,), j, jnp.int32)
            for k in range(NB // L):
                vals = slab_v[s, j, pl.ds(k * L, L)]
                plsc.store_scatter(t2p_v.at[s], [pidx[k], hofs[k] + jb], vals)

    fire(t0, 0)

    @pl.loop(0, 244, step=2)
    def _tile(tl):
        for s in range(2):
            t = t0 + tl + s

            @pl.when(tl + s + 1 < 244)
            def _():
                fire(t + 1, 1 - s)

            pltpu.make_async_copy(
                tabt_hbm.at[:, pl.ds(0, NB)], slab_v.at[s], rsem.at[s]
            ).wait()

            @pl.when(tl + s >= 2)
            def _():
                drain_write(s)

            transpose(s)

            pltpu.async_copy(
                t2p_v.at[s],
                out_hbm.at[pl.ds(t * (NB // 2), NB // 2)],
                wsem.at[s],
            )

    for s in range(2):
        drain_write(s)

    @pl.when(n_w > 244)
    def _():
        t = t0 + 244
        pltpu.sync_copy(tabt_hbm.at[:, pl.ds(t * NB, NB)], slab_v.at[0])
        transpose(0)
        pltpu.sync_copy(
            t2p_v.at[0], out_hbm.at[pl.ds(t * (NB // 2), NB // 2)]
        )

    @pl.when(wid == NW - 1)
    def _():
        pltpu.sync_copy(tail_hbm, tl_v)
        pltpu.sync_copy(tl_v, out_hbm.at[pl.ds(FULL_TILES * (NB // 2), TAIL // 2)])


def _gather_body(xt_hbm, tab_hbm, out_hbm, idx_v, rows_v, t_v, gsem, wsem):
    wid = lax.axis_index("s") * NC + lax.axis_index("c")
    g0 = wid * BLK_PER_W

    pltpu.sync_copy(xt_hbm.at[pl.ds(g0, BLK_PER_W)], idx_v)

    iota = lax.iota(jnp.int32, L)
    jtr = [(iota + k * L) // 8 for k in range(DIM // L)]
    jr = [(iota + k * L) % 8 for k in range(DIM // L)]

    def fire(g, s):
        pltpu.async_copy(tab_hbm.at[idx_v.at[g]], rows_v.at[s], gsem.at[s])

    def drain_writes(s):
        pltpu.make_async_copy(
            tab_hbm.at[pl.ds(0, NB)], rows_v.at[s], wsem.at[s]
        ).wait()

    for s in range(NSETS - 1):
        fire(s, s)

    @pl.loop(0, BLK_PER_W, step=NSETS)
    def _blk(gl):
        for s in range(NSETS):
            g = gl + s

            @pl.when(g + NSETS - 1 < BLK_PER_W)
            def _():
                fire(g + NSETS - 1, (s + NSETS - 1) % NSETS)

            pltpu.make_async_copy(
                tab_hbm.at[pl.ds(0, NB)], rows_v.at[s], gsem.at[s]
            ).wait()

            @pl.when(g >= NSETS)
            def _():
                drain_writes(s)

            @pl.loop(0, NB, unroll=4)
            def _c(c):
                cidx = jnp.full((L,), c, jnp.int32)
                for k in range(DIM // L):
                    vals = rows_v[s, c, pl.ds(k * L, L)]
                    plsc.store_scatter(t_v.at[s], [jtr[k], jr[k], cidx], vals)

            gg = g0 + g
            h = gg // 128
            bc = gg % 128
            pltpu.async_copy(
                t_v.at[s].at[:, :, pl.ds(0, NB)],
                out_hbm.at[h, :, bc],
                wsem.at[s],
            )

    for s in range(NSETS):
        drain_writes(s)


def kernel(x, table):
    xt = x.T.reshape(NBLK, NB).astype(jnp.int32)
    tabt = table.T
    tail = table[FULL_TILES * NB:].reshape(TAIL // 2, NB)
    mesh = plsc.VectorSubcoreMesh(core_axis_name="c", subcore_axis_name="s")

    build = pl.kernel(
        _build_body,
        out_type=jax.ShapeDtypeStruct((NROWS // 2, NB), jnp.float32),
        mesh=mesh,
        scratch_types=[
            pltpu.VMEM((2, DIM, NB), jnp.float32),
            pltpu.VMEM((2, DIM, NB), jnp.float32),
            pltpu.VMEM((TAIL // 2, NB), jnp.float32),
            pltpu.SemaphoreType.DMA((2,)),
            pltpu.SemaphoreType.DMA((2,)),
        ],
        compiler_params=pltpu.CompilerParams(
            use_tc_tiling_on_sc=True, needs_layout_passes=False
        ),
    )
    tab_pairs = build(tabt, tail)
    tab_lin = tab_pairs.reshape(NROWS, DIM)

    gather = pl.kernel(
        _gather_body,
        out_type=jax.ShapeDtypeStruct((HIST, 8, 128, 8, NB), jnp.float32),
        mesh=mesh,
        scratch_types=[
            pltpu.VMEM((BLK_PER_W, NB), jnp.int32),
            pltpu.VMEM((NSETS, NB, DIM), jnp.float32),
            pltpu.VMEM((NSETS, 8, 8, TSTRIDE), jnp.float32),
            pltpu.SemaphoreType.DMA((NSETS,)),
            pltpu.SemaphoreType.DMA((NSETS,)),
        ],
        compiler_params=pltpu.CompilerParams(
            use_tc_tiling_on_sc=False, needs_layout_passes=False
        ),
    )
    out5 = gather(xt, tab_lin)
    return jnp.transpose(out5, (2, 4, 0, 1, 3)).reshape(BATCH, HIST, DIM)


# speedup vs baseline: 1.6167x; 1.6167x over previous
import jax
import jax.numpy as jnp
from jax import lax
from jax.experimental import pallas as pl
from jax.experimental.pallas import tpu as pltpu
from jax.experimental.pallas import tpu_sc as plsc

BATCH = 16384
HIST = 50
DIM = 64
NB = 128
NBLK = HIST * (BATCH // NB)
NC = 2
NS = 16
NW = NC * NS
BLK_PER_W = NBLK // NW
L = 16
TSTRIDE = NB + 1
NSETS = 4


def _body(xt_hbm, tab_hbm, out_hbm, idx_v, rows_v, t_v, gsem, wsem):
    wid = lax.axis_index("s") * NC + lax.axis_index("c")
    g0 = wid * BLK_PER_W

    pltpu.sync_copy(xt_hbm.at[pl.ds(g0, BLK_PER_W)], idx_v)

    iota = lax.iota(jnp.int32, L)
    jtr = [(iota + k * L) // 8 for k in range(DIM // L)]
    jr = [(iota + k * L) % 8 for k in range(DIM // L)]

    def fire(g, s):
        pltpu.async_copy(tab_hbm.at[idx_v.at[g]], rows_v.at[s], gsem.at[s])

    def drain_writes(s):
        pltpu.make_async_copy(
            tab_hbm.at[pl.ds(0, NB)], rows_v.at[s], wsem.at[s]
        ).wait()

    for s in range(NSETS - 1):
        fire(s, s)

    @pl.loop(0, BLK_PER_W, step=NSETS)
    def _blk(gl):
        for s in range(NSETS):
            g = gl + s

            @pl.when(g + NSETS - 1 < BLK_PER_W)
            def _():
                fire(g + NSETS - 1, (s + NSETS - 1) % NSETS)

            pltpu.make_async_copy(
                tab_hbm.at[pl.ds(0, NB)], rows_v.at[s], gsem.at[s]
            ).wait()

            @pl.when(g >= NSETS)
            def _():
                drain_writes(s)

            @pl.loop(0, NB, unroll=8)
            def _c(c):
                cidx = jnp.full((L,), c, jnp.int32)
                for k in range(DIM // L):
                    vals = rows_v[s, c, pl.ds(k * L, L)]
                    plsc.store_scatter(t_v.at[s], [jtr[k], jr[k], cidx], vals)

            gg = g0 + g
            h = gg // 128
            bc = gg % 128
            pltpu.async_copy(
                t_v.at[s].at[:, :, pl.ds(0, NB)],
                out_hbm.at[h, :, bc],
                wsem.at[s],
            )

    for s in range(NSETS):
        drain_writes(s)


def kernel(x, table):
    xt = x.T.reshape(NBLK, NB).astype(jnp.int32)
    mesh = plsc.VectorSubcoreMesh(core_axis_name="c", subcore_axis_name="s")
    grab = pl.kernel(
        _body,
        out_type=jax.ShapeDtypeStruct((HIST, 8, 128, 8, NB), jnp.float32),
        mesh=mesh,
        scratch_types=[
            pltpu.VMEM((BLK_PER_W, NB), jnp.int32),
            pltpu.VMEM((NSETS, NB, DIM), jnp.float32),
            pltpu.VMEM((NSETS, 8, 8, TSTRIDE), jnp.float32),
            pltpu.SemaphoreType.DMA((NSETS,)),
            pltpu.SemaphoreType.DMA((NSETS,)),
        ],
        compiler_params=pltpu.CompilerParams(
            use_tc_tiling_on_sc=False, needs_layout_passes=False
        ),
    )
    out5 = grab(xt, table)
    return jnp.transpose(out5, (2, 4, 0, 1, 3)).reshape(BATCH, HIST, DIM)


# speedup vs baseline: 1.6309x; 1.0088x over previous
import jax
import jax.numpy as jnp
from jax import lax
from jax.experimental import pallas as pl
from jax.experimental.pallas import tpu as pltpu
from jax.experimental.pallas import tpu_sc as plsc

BATCH = 16384
HIST = 50
DIM = 64
NB = 128
NBLK = HIST * (BATCH // NB)
NC = 2
NS = 16
NW = NC * NS
BLK_PER_W = NBLK // NW
L = 16
TSTRIDE = NB + 1
NSETS = 4


def _body(xt_hbm, tab_hbm, out_hbm, idx_v, rows_v, t_v, gsem, wsem):
    wid = lax.axis_index("s") * NC + lax.axis_index("c")
    g0 = wid * BLK_PER_W

    pltpu.sync_copy(xt_hbm.at[pl.ds(g0, BLK_PER_W)], idx_v)

    iota = lax.iota(jnp.int32, L)
    jtr = [(iota + k * L) // 8 for k in range(DIM // L)]
    jr = [(iota + k * L) % 8 for k in range(DIM // L)]

    def fire(g, s):
        pltpu.async_copy(tab_hbm.at[idx_v.at[g]], rows_v.at[s], gsem.at[s])

    def drain_writes(s):
        pltpu.make_async_copy(
            tab_hbm.at[pl.ds(0, NB)], rows_v.at[s], wsem.at[s]
        ).wait()

    for s in range(NSETS - 1):
        fire(s, s)

    @pl.loop(0, BLK_PER_W, step=NSETS)
    def _blk(gl):
        for s in range(NSETS):
            g = gl + s

            @pl.when(g + NSETS - 1 < BLK_PER_W)
            def _():
                fire(g + NSETS - 1, (s + NSETS - 1) % NSETS)

            pltpu.make_async_copy(
                tab_hbm.at[pl.ds(0, NB)], rows_v.at[s], gsem.at[s]
            ).wait()

            @pl.when(g >= NSETS)
            def _():
                drain_writes(s)

            @pl.loop(0, NB, unroll=4)
            def _c(c):
                cidx = jnp.full((L,), c, jnp.int32)
                for k in range(DIM // L):
                    vals = rows_v[s, c, pl.ds(k * L, L)]
                    plsc.store_scatter(t_v.at[s], [jtr[k], jr[k], cidx], vals)

            gg = g0 + g
            h = gg // 128
            bc = gg % 128
            pltpu.async_copy(
                t_v.at[s].at[:, :, pl.ds(0, NB)],
                out_hbm.at[h, :, bc],
                wsem.at[s],
            )

    for s in range(NSETS):
        drain_writes(s)


def kernel(x, table):
    xt = x.T.reshape(NBLK, NB).astype(jnp.int32)
    mesh = plsc.VectorSubcoreMesh(core_axis_name="c", subcore_axis_name="s")
    grab = pl.kernel(
        _body,
        out_type=jax.ShapeDtypeStruct((HIST, 8, 128, 8, NB), jnp.float32),
        mesh=mesh,
        scratch_types=[
            pltpu.VMEM((BLK_PER_W, NB), jnp.int32),
            pltpu.VMEM((NSETS, NB, DIM), jnp.float32),
            pltpu.VMEM((NSETS, 8, 8, TSTRIDE), jnp.float32),
            pltpu.SemaphoreType.DMA((NSETS,)),
            pltpu.SemaphoreType.DMA((NSETS,)),
        ],
        compiler_params=pltpu.CompilerParams(
            use_tc_tiling_on_sc=False, needs_layout_passes=False
        ),
    )
    out5 = grab(xt, table)
    return jnp.transpose(out5, (2, 4, 0, 1, 3)).reshape(BATCH, HIST, DIM)
